# baseline (speedup 1.0000x reference)
import functools

import jax
import jax.numpy as jnp
from jax import lax
from jax.experimental import pallas as pl
from jax.experimental.pallas import tpu as pltpu
from jax.experimental.pallas import tpu_sc as plsc

B = 4096
L = 200
D = 128
C = 1000

NC, NS = 2, 16
NW = NC * NS
BPW = B // NW
C0, C1 = 128, 72

_LANES = 16
_G = D // _LANES


_UNROLL = 8


def _accum_rows(buf, n, acc):
    def body(i, acc):
        r0 = i * _UNROLL
        for u in range(_UNROLL):
            acc = tuple(acc[g] + buf[r0 + u, pl.ds(g * _LANES, _LANES)]
                        for g in range(_G))
        return acc
    return lax.fori_loop(0, n // _UNROLL, body, acc)


_NSETS = 3
B0 = 3584
B1 = B - B0


def _bag_sum_body(bags_hbm, table_hbm, out_hbm,
                  idx_v, buf00, buf01, buf10, buf11, buf20, buf21, outs_v,
                  sem00, sem01, sem10, sem11, sem20, sem21,
                  *, start, bpwh):
    wid = lax.axis_index("s") * NC + lax.axis_index("c")
    base = start + wid * bpwh
    bufs = ((buf00, buf01), (buf10, buf11), (buf20, buf21))
    sems = ((sem00, sem01), (sem10, sem11), (sem20, sem21))

    pltpu.sync_copy(bags_hbm.at[pl.ds(base * L, bpwh * L)], idx_v)

    def issue0(b, k):
        pltpu.async_copy(table_hbm.at[idx_v.at[pl.ds(b * L, C0)]],
                         bufs[k][0], sems[k][0])

    def issue1(b, k):
        pltpu.async_copy(table_hbm.at[idx_v.at[pl.ds(b * L + C0, C1)]],
                         bufs[k][1], sems[k][1])

    def drain_reduce(b, k):
        acc = tuple(jnp.zeros((_LANES,), jnp.float32) for _ in range(_G))
        pltpu.make_async_copy(
            table_hbm.at[idx_v.at[pl.ds(b * L, C0)]],
            bufs[k][0], sems[k][0]).wait()
        acc = _accum_rows(bufs[k][0], C0, acc)

        @pl.when(b + _NSETS < bpwh)
        def _():
            issue0(b + _NSETS, k)

        pltpu.make_async_copy(
            table_hbm.at[idx_v.at[pl.ds(b * L + C0, C1)]],
            bufs[k][1], sems[k][1]).wait()
        acc = _accum_rows(bufs[k][1], C1, acc)

        @pl.when(b + _NSETS < bpwh)
        def _():
            issue1(b + _NSETS, k)

        for g in range(_G):
            outs_v[b, pl.ds(g * _LANES, _LANES)] = acc[g]

    for k in range(_NSETS):
        issue0(k, k)
        issue1(k, k)

    def group_body(i, _):
        for k in range(_NSETS):
            drain_reduce(_NSETS * i + k, k)
        return 0

    ngroups = bpwh // _NSETS
    lax.fori_loop(0, ngroups, group_body, 0)
    for k in range(bpwh - _NSETS * ngroups):
        drain_reduce(_NSETS * ngroups + k, k)

    pltpu.sync_copy(outs_v, out_hbm.at[pl.ds(wid * bpwh, bpwh)])


def _sc_bag_sum_chunk(input_bags_flat, emb_table, start, nbags):
    bpwh = nbags // NW
    mesh = plsc.VectorSubcoreMesh(core_axis_name="c", subcore_axis_name="s")
    return pl.kernel(
        functools.partial(_bag_sum_body, start=start, bpwh=bpwh),
        out_type=jax.ShapeDtypeStruct((nbags, D), jnp.float32),
        mesh=mesh,
        scratch_types=(
            [pltpu.VMEM((bpwh * L,), jnp.int32)]
            + [pltpu.VMEM((n, D), jnp.float32)
               for _ in range(_NSETS) for n in (C0, C1)]
            + [pltpu.VMEM((bpwh, D), jnp.float32)]
            + [pltpu.SemaphoreType.DMA] * (2 * _NSETS)
        ),
    )(input_bags_flat, emb_table)


_BLK = 256


def _head_body(h_ref, w_ref, bt_ref, o_ref):
    h = h_ref[...] * (1.0 / L)
    logits = jax.lax.dot_general(
        w_ref[...], h, (((1,), (1,)), ((), ())),
        preferred_element_type=jnp.float32) + bt_ref[...]
    m = jnp.max(logits, axis=0, keepdims=True)
    s = logits - m
    lse = jnp.log(jnp.sum(jnp.exp(s), axis=0, keepdims=True))
    o_ref[...] = s - lse


def _head_body2(prev_ref, h_ref, w_ref, bt_ref, o_ref):
    del prev_ref
    _head_body(h_ref, w_ref, bt_ref, o_ref)


def _tc_head_first(hidden_sums, W, bt):
    return pl.pallas_call(
        _head_body,
        grid=(B0 // _BLK,),
        in_specs=[
            pl.BlockSpec((_BLK, D), lambda i: (i, 0)),
            pl.BlockSpec((C, D), lambda i: (0, 0)),
            pl.BlockSpec((C, 1), lambda i: (0, 0)),
        ],
        out_specs=pl.BlockSpec((C, _BLK), lambda i: (0, i)),
        out_shape=jax.ShapeDtypeStruct((C, B), jnp.float32),
    )(hidden_sums, W, bt)


def _tc_head_second(prev, hidden_sums, W, bt):
    nblk0 = B0 // _BLK
    return pl.pallas_call(
        _head_body2,
        grid=(B1 // _BLK,),
        in_specs=[
            pl.BlockSpec(memory_space=pltpu.MemorySpace.HBM),
            pl.BlockSpec((_BLK, D), lambda i: (i, 0)),
            pl.BlockSpec((C, D), lambda i: (0, 0)),
            pl.BlockSpec((C, 1), lambda i: (0, 0)),
        ],
        out_specs=pl.BlockSpec((C, _BLK), lambda i: (0, i + nblk0)),
        out_shape=jax.ShapeDtypeStruct((C, B), jnp.float32),
        input_output_aliases={0: 0},
    )(prev, hidden_sums, W, bt)


def kernel(input_bags, emb_table, W, b):
    flat = input_bags.astype(jnp.int32).reshape(-1)
    bt = b.reshape(C, 1)
    s0 = _sc_bag_sum_chunk(flat, emb_table, 0, B0)
    s1 = _sc_bag_sum_chunk(flat, emb_table, B0, B1)
    o = _tc_head_first(s0, W, bt)
    o = _tc_head_second(o, s1, W, bt)
    return o.T


# speedup vs baseline: 1.0159x; 1.0159x over previous
import functools

import jax
import jax.numpy as jnp
from jax import lax
from jax.experimental import pallas as pl
from jax.experimental.pallas import tpu as pltpu
from jax.experimental.pallas import tpu_sc as plsc

B = 4096
L = 200
D = 128
C = 1000

NC, NS = 2, 16
NW = NC * NS
BPW = B // NW
C0, C1 = 104, 96

_LANES = 16
_G = D // _LANES


_UNROLL = 4


def _accum_rows(buf, n, acc):
    def body(i, acc):
        r0 = i * _UNROLL
        for u in range(_UNROLL):
            acc = tuple(acc[g] + buf[r0 + u, pl.ds(g * _LANES, _LANES)]
                        for g in range(_G))
        return acc
    return lax.fori_loop(0, n // _UNROLL, body, acc)


_NSETS = 3
B0 = 3584
B1 = B - B0


def _bag_sum_body(bags_hbm, table_hbm, out_hbm,
                  idx_v, buf00, buf01, buf10, buf11, buf20, buf21, outs_v,
                  sem00, sem01, sem10, sem11, sem20, sem21,
                  *, start, bpwh):
    wid = lax.axis_index("s") * NC + lax.axis_index("c")
    base = start + wid * bpwh
    bufs = ((buf00, buf01), (buf10, buf11), (buf20, buf21))
    sems = ((sem00, sem01), (sem10, sem11), (sem20, sem21))

    pltpu.sync_copy(bags_hbm.at[pl.ds(base * L, bpwh * L)], idx_v)

    def issue0(b, k):
        pltpu.async_copy(table_hbm.at[idx_v.at[pl.ds(b * L, C0)]],
                         bufs[k][0], sems[k][0])

    def issue1(b, k):
        pltpu.async_copy(table_hbm.at[idx_v.at[pl.ds(b * L + C0, C1)]],
                         bufs[k][1], sems[k][1])

    def drain_reduce(b, k):
        acc = tuple(jnp.zeros((_LANES,), jnp.float32) for _ in range(_G))
        pltpu.make_async_copy(
            table_hbm.at[idx_v.at[pl.ds(b * L, C0)]],
            bufs[k][0], sems[k][0]).wait()
        acc = _accum_rows(bufs[k][0], C0, acc)

        @pl.when(b + _NSETS < bpwh)
        def _():
            issue0(b + _NSETS, k)

        pltpu.make_async_copy(
            table_hbm.at[idx_v.at[pl.ds(b * L + C0, C1)]],
            bufs[k][1], sems[k][1]).wait()
        acc = _accum_rows(bufs[k][1], C1, acc)

        @pl.when(b + _NSETS < bpwh)
        def _():
            issue1(b + _NSETS, k)

        for g in range(_G):
            outs_v[b, pl.ds(g * _LANES, _LANES)] = acc[g]

    for k in range(_NSETS):
        issue0(k, k)
        issue1(k, k)

    def group_body(i, _):
        for k in range(_NSETS):
            drain_reduce(_NSETS * i + k, k)
        return 0

    ngroups = bpwh // _NSETS
    lax.fori_loop(0, ngroups, group_body, 0)
    for k in range(bpwh - _NSETS * ngroups):
        drain_reduce(_NSETS * ngroups + k, k)

    pltpu.sync_copy(outs_v, out_hbm.at[pl.ds(wid * bpwh, bpwh)])


def _sc_bag_sum_chunk(input_bags_flat, emb_table, start, nbags):
    bpwh = nbags // NW
    mesh = plsc.VectorSubcoreMesh(core_axis_name="c", subcore_axis_name="s")
    return pl.kernel(
        functools.partial(_bag_sum_body, start=start, bpwh=bpwh),
        out_type=jax.ShapeDtypeStruct((nbags, D), jnp.float32),
        mesh=mesh,
        scratch_types=(
            [pltpu.VMEM((bpwh * L,), jnp.int32)]
            + [pltpu.VMEM((n, D), jnp.float32)
               for _ in range(_NSETS) for n in (C0, C1)]
            + [pltpu.VMEM((bpwh, D), jnp.float32)]
            + [pltpu.SemaphoreType.DMA] * (2 * _NSETS)
        ),
    )(input_bags_flat, emb_table)


_BLK = 256


def _head_body(h_ref, w_ref, bt_ref, o_ref):
    h = h_ref[...] * (1.0 / L)
    logits = jax.lax.dot_general(
        w_ref[...], h, (((1,), (1,)), ((), ())),
        preferred_element_type=jnp.float32) + bt_ref[...]
    m = jnp.max(logits, axis=0, keepdims=True)
    s = logits - m
    lse = jnp.log(jnp.sum(jnp.exp(s), axis=0, keepdims=True))
    o_ref[...] = s - lse


def _head_body2(prev_ref, h_ref, w_ref, bt_ref, o_ref):
    del prev_ref
    _head_body(h_ref, w_ref, bt_ref, o_ref)


def _tc_head_first(hidden_sums, W, bt):
    return pl.pallas_call(
        _head_body,
        grid=(B0 // _BLK,),
        in_specs=[
            pl.BlockSpec((_BLK, D), lambda i: (i, 0)),
            pl.BlockSpec((C, D), lambda i: (0, 0)),
            pl.BlockSpec((C, 1), lambda i: (0, 0)),
        ],
        out_specs=pl.BlockSpec((C, _BLK), lambda i: (0, i)),
        out_shape=jax.ShapeDtypeStruct((C, B), jnp.float32),
    )(hidden_sums, W, bt)


def _tc_head_second(prev, hidden_sums, W, bt):
    nblk0 = B0 // _BLK
    return pl.pallas_call(
        _head_body2,
        grid=(B1 // _BLK,),
        in_specs=[
            pl.BlockSpec(memory_space=pltpu.MemorySpace.HBM),
            pl.BlockSpec((_BLK, D), lambda i: (i, 0)),
            pl.BlockSpec((C, D), lambda i: (0, 0)),
            pl.BlockSpec((C, 1), lambda i: (0, 0)),
        ],
        out_specs=pl.BlockSpec((C, _BLK), lambda i: (0, i + nblk0)),
        out_shape=jax.ShapeDtypeStruct((C, B), jnp.float32),
        input_output_aliases={0: 0},
    )(prev, hidden_sums, W, bt)


def kernel(input_bags, emb_table, W, b):
    flat = input_bags.astype(jnp.int32).reshape(-1)
    bt = b.reshape(C, 1)
    s0 = _sc_bag_sum_chunk(flat, emb_table, 0, B0)
    s1 = _sc_bag_sum_chunk(flat, emb_table, B0, B1)
    o = _tc_head_first(s0, W, bt)
    o = _tc_head_second(o, s1, W, bt)
    return o.T
